# baseline (speedup 1.0000x reference)
import functools

import jax
import jax.numpy as jnp
from jax import lax
from jax.experimental import pallas as pl
from jax.experimental.pallas import tpu as pltpu
from jax.experimental.pallas import tpu_sc as plsc

H = 384
W = 384
C = 192
NPOS = 31
NSEG = NPOS + 1
HB = 64

NW = 32
BPW = H * C // NW
CH = 64
NCH = BPW // CH


def _reduce_kernel(hp_ref, vp_ref, x_ref, colexp_ref, pix_ref):
    h = pl.program_id(0)
    nsteps = pl.num_programs(0)
    col_i = lax.broadcasted_iota(jnp.int32, (HB, 1), 0) + h * HB
    row_i = lax.broadcasted_iota(jnp.int32, (1, HB), 1) + h * HB
    acc_c = jnp.zeros((HB, 1), jnp.int32)
    acc_r = jnp.zeros((1, HB), jnp.int32)
    for k in range(NPOS):
        p = hp_ref[0, k]
        acc_c += (p <= col_i).astype(jnp.int32)
        acc_r += (p <= row_i).astype(jnp.int32)
    pix_ref[...] = acc_c * C + lax.broadcasted_iota(jnp.int32, (HB, C), 1)
    onehot_t = (acc_r == lax.broadcasted_iota(jnp.int32, (NSEG, HB), 0)
                ).astype(jnp.float32)
    part = lax.dot_general(onehot_t, x_ref[...], (((1,), (0,)), ((), ())),
                           preferred_element_type=jnp.float32)

    @pl.when(h == 0)
    def _():
        colexp_ref[...] = part

    @pl.when(h > 0)
    def _():
        colexp_ref[...] += part

    @pl.when(h == nsteps - 1)
    def _():
        jj_r = lax.broadcasted_iota(jnp.int32, (1, W), 1)
        jj_c = lax.broadcasted_iota(jnp.int32, (W, 1), 0)
        acc_jr = jnp.zeros((1, W), jnp.int32)
        acc_jc = jnp.zeros((W, 1), jnp.int32)
        for k in range(NPOS):
            p = vp_ref[0, k]
            acc_jr += (p <= jj_r).astype(jnp.int32)
            acc_jc += (p <= jj_c).astype(jnp.int32)
        ohct = (acc_jr == lax.broadcasted_iota(jnp.int32, (NSEG, W), 0)
                ).astype(jnp.float32)
        ohc = (acc_jc == lax.broadcasted_iota(jnp.int32, (W, NSEG), 1)
               ).astype(jnp.float32)
        cnt = jnp.sum(ohc, axis=0, keepdims=True)
        ohc_s = ohc * (1.0 / jnp.maximum(cnt, 1.0))
        for r in range(NSEG):
            lo = hp_ref[0, r - 1] if r > 0 else 0
            hi = hp_ref[0, r] if r < NPOS else H
            rs = 1.0 / jnp.maximum(hi - lo, 1).astype(jnp.float32)
            mean_r = lax.dot_general(
                colexp_ref[r], ohc_s, (((1,), (0,)), ((), ())),
                preferred_element_type=jnp.float32)
            colexp_ref[r] = lax.dot_general(
                mean_r * rs, ohct, (((1,), (0,)), ((), ())),
                preferred_element_type=jnp.float32)


def _sc_gather_kernel(table_hbm, idx_hbm, out_hbm, idx_v, rows_v, sem):
    wid = lax.axis_index("s") * 2 + lax.axis_index("c")

    def body(g, carry):
        base = wid * BPW + g * CH
        pltpu.sync_copy(idx_hbm.at[pl.ds(base, CH)], idx_v)
        pltpu.async_copy(table_hbm.at[idx_v], rows_v, sem).wait()
        pltpu.sync_copy(rows_v, out_hbm.at[pl.ds(base, CH)])
        return carry

    lax.fori_loop(0, NCH, body, 0)


def kernel(input, h_positions, v_positions):
    xt = jnp.transpose(input[0], (0, 2, 1))
    hp = h_positions.astype(jnp.int32).reshape(1, NPOS)
    vp = v_positions.astype(jnp.int32).reshape(1, NPOS)

    colexp, pix = pl.pallas_call(
        _reduce_kernel,
        grid=(H // HB,),
        in_specs=[
            pl.BlockSpec(memory_space=pltpu.SMEM),
            pl.BlockSpec(memory_space=pltpu.SMEM),
            pl.BlockSpec((HB, C, W), lambda h: (h, 0, 0)),
        ],
        out_specs=[
            pl.BlockSpec((NSEG, C, W), lambda h: (0, 0, 0)),
            pl.BlockSpec((HB, C), lambda h: (h, 0)),
        ],
        out_shape=[
            jax.ShapeDtypeStruct((NSEG, C, W), jnp.float32),
            jax.ShapeDtypeStruct((H, C), jnp.int32),
        ],
    )(hp, vp, xt)

    sc_gather = pl.kernel(
        _sc_gather_kernel,
        out_type=jax.ShapeDtypeStruct((H * C, W), jnp.float32),
        mesh=plsc.VectorSubcoreMesh(core_axis_name="c", subcore_axis_name="s"),
        scratch_types=[
            pltpu.VMEM((CH,), jnp.int32),
            pltpu.VMEM((CH, W), jnp.float32),
            pltpu.SemaphoreType.DMA,
        ],
    )
    yflat = sc_gather(colexp.reshape(NSEG * C, W), pix.reshape(H * C))

    return jnp.transpose(yflat.reshape(H, C, W), (0, 2, 1))[None]


# speedup vs baseline: 1.1921x; 1.1921x over previous
import functools

import jax
import jax.numpy as jnp
from jax import lax
from jax.experimental import pallas as pl
from jax.experimental.pallas import tpu as pltpu
from jax.experimental.pallas import tpu_sc as plsc

H = 384
W = 384
C = 192
NPOS = 31
NSEG = NPOS + 1
HB = 64

NW = 32
BPW = H * C // NW
CH = 64
NCH = BPW // CH


def _reduce_kernel(hp_ref, vp_ref, x_ref, colexp_ref, pix_ref):
    h = pl.program_id(0)
    nsteps = pl.num_programs(0)
    col_i = lax.broadcasted_iota(jnp.int32, (HB, 1), 0) + h * HB
    row_i = lax.broadcasted_iota(jnp.int32, (1, HB), 1) + h * HB
    acc_c = jnp.zeros((HB, 1), jnp.int32)
    acc_r = jnp.zeros((1, HB), jnp.int32)
    for k in range(NPOS):
        p = hp_ref[0, k]
        acc_c += (p <= col_i).astype(jnp.int32)
        acc_r += (p <= row_i).astype(jnp.int32)
    pix_ref[...] = acc_c * C + lax.broadcasted_iota(jnp.int32, (HB, C), 1)
    onehot_t = (acc_r == lax.broadcasted_iota(jnp.int32, (NSEG, HB), 0)
                ).astype(jnp.float32)
    part = lax.dot_general(onehot_t, x_ref[...], (((1,), (0,)), ((), ())),
                           preferred_element_type=jnp.float32)

    @pl.when(h == 0)
    def _():
        colexp_ref[...] = part

    @pl.when(h > 0)
    def _():
        colexp_ref[...] += part

    @pl.when(h == nsteps - 1)
    def _():
        jj_r = lax.broadcasted_iota(jnp.int32, (1, W), 1)
        jj_c = lax.broadcasted_iota(jnp.int32, (W, 1), 0)
        acc_jr = jnp.zeros((1, W), jnp.int32)
        acc_jc = jnp.zeros((W, 1), jnp.int32)
        for k in range(NPOS):
            p = vp_ref[0, k]
            acc_jr += (p <= jj_r).astype(jnp.int32)
            acc_jc += (p <= jj_c).astype(jnp.int32)
        ohct = (acc_jr == lax.broadcasted_iota(jnp.int32, (NSEG, W), 0)
                ).astype(jnp.float32)
        ohc = (acc_jc == lax.broadcasted_iota(jnp.int32, (W, NSEG), 1)
               ).astype(jnp.float32)
        cnt = jnp.sum(ohc, axis=0, keepdims=True)
        ohc_s = ohc * (1.0 / jnp.maximum(cnt, 1.0))
        for r in range(NSEG):
            lo = hp_ref[0, r - 1] if r > 0 else 0
            hi = hp_ref[0, r] if r < NPOS else H
            rs = 1.0 / jnp.maximum(hi - lo, 1).astype(jnp.float32)
            mean_r = lax.dot_general(
                colexp_ref[r], ohc_s, (((1,), (0,)), ((), ())),
                preferred_element_type=jnp.float32)
            colexp_ref[r] = lax.dot_general(
                mean_r * rs, ohct, (((1,), (0,)), ((), ())),
                preferred_element_type=jnp.float32)


def _sc_gather_kernel(table_hbm, idx_hbm, out_hbm, idx_v, rows0_v, rows1_v,
                      sem0, sem1):
    wid = lax.axis_index("s") * 2 + lax.axis_index("c")
    wbase = wid * BPW
    pltpu.sync_copy(idx_hbm.at[pl.ds(wbase, BPW)], idx_v)

    def body(gp, carry):
        b0 = gp * (2 * CH)
        h0 = pltpu.async_copy(table_hbm.at[idx_v.at[pl.ds(b0, CH)]],
                              rows0_v, sem0)
        h1 = pltpu.async_copy(table_hbm.at[idx_v.at[pl.ds(b0 + CH, CH)]],
                              rows1_v, sem1)
        h0.wait()
        pltpu.sync_copy(rows0_v, out_hbm.at[pl.ds(wbase + b0, CH)])
        h1.wait()
        pltpu.sync_copy(rows1_v, out_hbm.at[pl.ds(wbase + b0 + CH, CH)])
        return carry

    lax.fori_loop(0, NCH // 2, body, 0)


def kernel(input, h_positions, v_positions):
    xt = jnp.transpose(input[0], (0, 2, 1))
    hp = h_positions.astype(jnp.int32).reshape(1, NPOS)
    vp = v_positions.astype(jnp.int32).reshape(1, NPOS)

    colexp, pix = pl.pallas_call(
        _reduce_kernel,
        grid=(H // HB,),
        in_specs=[
            pl.BlockSpec(memory_space=pltpu.SMEM),
            pl.BlockSpec(memory_space=pltpu.SMEM),
            pl.BlockSpec((HB, C, W), lambda h: (h, 0, 0)),
        ],
        out_specs=[
            pl.BlockSpec((NSEG, C, W), lambda h: (0, 0, 0)),
            pl.BlockSpec((HB, C), lambda h: (h, 0)),
        ],
        out_shape=[
            jax.ShapeDtypeStruct((NSEG, C, W), jnp.float32),
            jax.ShapeDtypeStruct((H, C), jnp.int32),
        ],
    )(hp, vp, xt)

    sc_gather = pl.kernel(
        _sc_gather_kernel,
        out_type=jax.ShapeDtypeStruct((H * C, W), jnp.float32),
        mesh=plsc.VectorSubcoreMesh(core_axis_name="c", subcore_axis_name="s"),
        scratch_types=[
            pltpu.VMEM((BPW,), jnp.int32),
            pltpu.VMEM((CH, W), jnp.float32),
            pltpu.VMEM((CH, W), jnp.float32),
            pltpu.SemaphoreType.DMA,
            pltpu.SemaphoreType.DMA,
        ],
    )
    yflat = sc_gather(colexp.reshape(NSEG * C, W), pix.reshape(H * C))

    return jnp.transpose(yflat.reshape(H, C, W), (0, 2, 1))[None]


# speedup vs baseline: 1.9519x; 1.6373x over previous
import jax
import jax.numpy as jnp
from jax import lax
from jax.experimental import pallas as pl
from jax.experimental.pallas import tpu as pltpu

H = 384
W = 384
C = 192
NPOS = 31
NSEG = NPOS + 1
HB = 64
IB = 32


def _reduce_kernel(hp_ref, vp_ref, x_ref, colexp_ref, ridx_ref):
    h = pl.program_id(0)
    nsteps = pl.num_programs(0)
    col_i = lax.broadcasted_iota(jnp.int32, (HB, 1), 0) + h * HB
    row_i = lax.broadcasted_iota(jnp.int32, (1, HB), 1) + h * HB
    acc_c = jnp.zeros((HB, 1), jnp.int32)
    acc_r = jnp.zeros((1, HB), jnp.int32)
    for k in range(NPOS):
        p = hp_ref[0, k]
        acc_c += (p <= col_i).astype(jnp.int32)
        acc_r += (p <= row_i).astype(jnp.int32)
    ridx_ref[...] = acc_c
    onehot_t = (acc_r == lax.broadcasted_iota(jnp.int32, (NSEG, HB), 0)
                ).astype(jnp.float32)
    part = lax.dot_general(onehot_t, x_ref[...], (((1,), (0,)), ((), ())),
                           preferred_element_type=jnp.float32)

    @pl.when(h == 0)
    def _():
        colexp_ref[...] = part

    @pl.when(h > 0)
    def _():
        colexp_ref[...] += part

    @pl.when(h == nsteps - 1)
    def _():
        jj_r = lax.broadcasted_iota(jnp.int32, (1, W), 1)
        jj_c = lax.broadcasted_iota(jnp.int32, (W, 1), 0)
        acc_jr = jnp.zeros((1, W), jnp.int32)
        acc_jc = jnp.zeros((W, 1), jnp.int32)
        for k in range(NPOS):
            p = vp_ref[0, k]
            acc_jr += (p <= jj_r).astype(jnp.int32)
            acc_jc += (p <= jj_c).astype(jnp.int32)
        ohct = (acc_jr == lax.broadcasted_iota(jnp.int32, (NSEG, W), 0)
                ).astype(jnp.float32)
        ohc = (acc_jc == lax.broadcasted_iota(jnp.int32, (W, NSEG), 1)
               ).astype(jnp.float32)
        cnt = jnp.sum(ohc, axis=0, keepdims=True)
        ohc_s = ohc * (1.0 / jnp.maximum(cnt, 1.0))
        for r in range(NSEG):
            lo = hp_ref[0, r - 1] if r > 0 else 0
            hi = hp_ref[0, r] if r < NPOS else H
            rs = 1.0 / jnp.maximum(hi - lo, 1).astype(jnp.float32)
            mean_r = lax.dot_general(
                colexp_ref[r], ohc_s, (((1,), (0,)), ((), ())),
                preferred_element_type=jnp.float32)
            colexp_ref[r] = lax.dot_general(
                mean_r * rs, ohct, (((1,), (0,)), ((), ())),
                preferred_element_type=jnp.float32)


def _row_gather_kernel(ridx_ref, colexp_ref, out_ref):
    base = pl.program_id(0) * IB

    def body(ii, carry):
        r = ridx_ref[base + ii]
        out_ref[pl.ds(ii, 1)] = colexp_ref[pl.ds(r, 1)]
        return carry

    lax.fori_loop(0, IB, body, 0)


def kernel(input, h_positions, v_positions):
    xt = jnp.transpose(input[0], (0, 2, 1))
    hp = h_positions.astype(jnp.int32).reshape(1, NPOS)
    vp = v_positions.astype(jnp.int32).reshape(1, NPOS)

    colexp, ridx = pl.pallas_call(
        _reduce_kernel,
        grid=(H // HB,),
        in_specs=[
            pl.BlockSpec(memory_space=pltpu.SMEM),
            pl.BlockSpec(memory_space=pltpu.SMEM),
            pl.BlockSpec((HB, C, W), lambda h: (h, 0, 0)),
        ],
        out_specs=[
            pl.BlockSpec((NSEG, C, W), lambda h: (0, 0, 0)),
            pl.BlockSpec((HB, 1), lambda h: (h, 0)),
        ],
        out_shape=[
            jax.ShapeDtypeStruct((NSEG, C, W), jnp.float32),
            jax.ShapeDtypeStruct((H, 1), jnp.int32),
        ],
    )(hp, vp, xt)

    yt = pl.pallas_call(
        _row_gather_kernel,
        grid=(H // IB,),
        in_specs=[
            pl.BlockSpec(memory_space=pltpu.SMEM),
            pl.BlockSpec((NSEG, C, W), lambda h: (0, 0, 0)),
        ],
        out_specs=pl.BlockSpec((IB, C, W), lambda h: (h, 0, 0)),
        out_shape=jax.ShapeDtypeStruct((H, C, W), jnp.float32),
    )(ridx.reshape(H), colexp)

    return jnp.transpose(yt, (0, 2, 1))[None]
